# baseline (speedup 1.0000x reference)
import functools

import jax
import jax.numpy as jnp
from jax import lax
from jax.experimental import pallas as pl
from jax.experimental.pallas import tpu as pltpu
from jax.experimental.pallas import tpu_sc as plsc

_B = 16384
_D = 32
_NE = 1_000_000
_NR = 1000
_L = 16

_EBLK = 32768
_EGRID = -(-_NE // _EBLK)
_PROWS = _EGRID * (_EBLK // 4)
_PRELR = 256


def _pack_body(x_ref, o_ref):
    ngrp = x_ref.shape[1] // 512
    for g in range(ngrp):
        y = x_ref[:, g * 512:(g + 1) * 512].T
        for j in range(4):
            o_ref[g * 128:(g + 1) * 128, j * _D:(j + 1) * _D] = (
                y[j * 128:(j + 1) * 128])


@functools.lru_cache(maxsize=None)
def _packers():
    ent = pl.pallas_call(
        _pack_body,
        grid=(_EGRID,),
        in_specs=[pl.BlockSpec((_D, _EBLK), lambda i: (0, i))],
        out_specs=pl.BlockSpec((_EBLK // 4, 128), lambda i: (i, 0)),
        out_shape=jax.ShapeDtypeStruct((_PROWS, 128), jnp.float32),
    )
    rel = pl.pallas_call(
        _pack_body,
        grid=(2,),
        in_specs=[pl.BlockSpec((_D, 512), lambda i: (0, i))],
        out_specs=pl.BlockSpec((128, 128), lambda i: (i, 0)),
        out_shape=jax.ShapeDtypeStruct((_PRELR, 128), jnp.float32),
    )
    return ent, rel


@functools.lru_cache(maxsize=None)
def _gather_kernel():
    info = plsc.get_sparse_core_info()
    nc, ns = info.num_cores, info.num_subcores
    nw = nc * ns
    bw = _B // nw
    ng = bw // _L
    mesh = plsc.VectorSubcoreMesh(core_axis_name="c", subcore_axis_name="s")

    @functools.partial(
        pl.kernel,
        mesh=mesh,
        compiler_params=pltpu.CompilerParams(use_tc_tiling_on_sc=True,
                                             needs_layout_passes=False),
        out_type=(
            jax.ShapeDtypeStruct((_D, _B), jnp.float32),
            jax.ShapeDtypeStruct((_D, _B), jnp.float32),
            jax.ShapeDtypeStruct((_D, _B), jnp.float32),
        ),
        scratch_types=[
            pltpu.VMEM((bw,), jnp.int32),
            pltpu.VMEM((bw,), jnp.int32),
            pltpu.VMEM((bw, 128), jnp.float32),
            pltpu.VMEM((4, _D, 128), jnp.float32),
            pltpu.SemaphoreType.DMA,
        ],
    )
    def k(head_hbm, rel_hbm, tail_hbm, entp_hbm, relp_hbm,
          out_h, out_r, out_t, idxv, rowv, grows, stage, sem):
        wid = lax.axis_index("s") * nc + lax.axis_index("c")
        base = wid * bw
        lane = lax.iota(jnp.int32, _L)
        for src_hbm, tbl_hbm, out_hbm in ((head_hbm, entp_hbm, out_h),
                                          (rel_hbm, relp_hbm, out_r),
                                          (tail_hbm, entp_hbm, out_t)):
            pltpu.sync_copy(src_hbm.at[pl.ds(base, bw)], idxv)
            for c in range(ng):
                s = pl.ds(c * _L, _L)
                e = idxv[s]
                rowv[s] = (
                    jax.lax.shift_left(
                        jax.lax.shift_right_logical(e, 9), 7)
                    + (e & 127))
            cps = [pltpu.async_copy(
                       tbl_hbm.at[rowv.at[pl.ds(j * 128, 128)]],
                       grows.at[pl.ds(j * 128, 128)], sem)
                   for j in range(4)]
            for cp in cps:
                cp.wait()

            @pl.loop(0, ng)
            def _grp(c):
                s = pl.ds(pl.multiple_of(c * _L, _L), _L)
                off = (jax.lax.shift_right_logical(idxv[s], 7) & 3) * _D
                bvec = c * _L + lane
                cbv = jnp.full((_L,), c // 8, dtype=jnp.int32)
                lpos = (c % 8) * _L + lane
                for kd in range(_D):
                    v = plsc.load_gather(grows, [bvec, off + kd])
                    plsc.store_scatter(
                        stage,
                        [cbv, jnp.full((_L,), kd, dtype=jnp.int32), lpos],
                        v)

            for cb in range(4):
                pltpu.sync_copy(
                    stage.at[cb],
                    out_hbm.at[:, pl.ds(base + cb * 128, 128)])

    return k


def kernel(head, rel, tail, entity_table, relation_table):
    pack_ent, pack_rel = _packers()
    entp = pack_ent(entity_table.T)
    relp = pack_rel(relation_table.T)
    out_h, out_r, out_t = _gather_kernel()(head, rel, tail, entp, relp)
    return (out_h.T, out_r.T, out_t.T)


# speedup vs baseline: 1.4275x; 1.4275x over previous
import functools

import jax
import jax.numpy as jnp
from jax import lax
from jax.experimental import pallas as pl
from jax.experimental.pallas import tpu as pltpu
from jax.experimental.pallas import tpu_sc as plsc

_B = 16384
_D = 32
_NE = 1_000_000
_NR = 1000
_L = 16

_EBLK = 32768
_EGRID = -(-_NE // _EBLK)
_PROWS = _EGRID * (_EBLK // 4)
_PRELR = 256


def _pack_body(x_ref, o_ref):
    ngrp = x_ref.shape[1] // 512
    sel = [jnp.eye(_D, 128, 32 * j, dtype=jnp.float32) for j in range(4)]
    for g in range(ngrp):
        acc = None
        for j in range(4):
            xj = x_ref[:, g * 512 + j * 128:g * 512 + (j + 1) * 128]
            t = jax.lax.dot_general(
                xj, sel[j], (((0,), (0,)), ((), ())),
                preferred_element_type=jnp.float32)
            acc = t if acc is None else acc + t
        o_ref[g * 128:(g + 1) * 128, :] = acc


@functools.lru_cache(maxsize=None)
def _packers():
    ent = pl.pallas_call(
        _pack_body,
        grid=(_EGRID,),
        in_specs=[pl.BlockSpec((_D, _EBLK), lambda i: (0, i))],
        out_specs=pl.BlockSpec((_EBLK // 4, 128), lambda i: (i, 0)),
        out_shape=jax.ShapeDtypeStruct((_PROWS, 128), jnp.float32),
    )
    rel = pl.pallas_call(
        _pack_body,
        grid=(2,),
        in_specs=[pl.BlockSpec((_D, 512), lambda i: (0, i))],
        out_specs=pl.BlockSpec((128, 128), lambda i: (i, 0)),
        out_shape=jax.ShapeDtypeStruct((_PRELR, 128), jnp.float32),
    )
    return ent, rel


@functools.lru_cache(maxsize=None)
def _gather_kernel():
    info = plsc.get_sparse_core_info()
    nc, ns = info.num_cores, info.num_subcores
    nw = nc * ns
    bw = _B // nw
    ng = bw // _L
    mesh = plsc.VectorSubcoreMesh(core_axis_name="c", subcore_axis_name="s")

    @functools.partial(
        pl.kernel,
        mesh=mesh,
        compiler_params=pltpu.CompilerParams(use_tc_tiling_on_sc=True,
                                             needs_layout_passes=False),
        out_type=(
            jax.ShapeDtypeStruct((_D, _B), jnp.float32),
            jax.ShapeDtypeStruct((_D, _B), jnp.float32),
            jax.ShapeDtypeStruct((_D, _B), jnp.float32),
        ),
        scratch_types=[
            pltpu.VMEM((bw,), jnp.int32),
            pltpu.VMEM((bw,), jnp.int32),
            pltpu.VMEM((bw, 128), jnp.float32),
            pltpu.VMEM((4, _D, 128), jnp.float32),
            pltpu.SemaphoreType.DMA,
        ],
    )
    def k(head_hbm, rel_hbm, tail_hbm, entp_hbm, relp_hbm,
          out_h, out_r, out_t, idxv, rowv, grows, stage, sem):
        wid = lax.axis_index("s") * nc + lax.axis_index("c")
        base = wid * bw
        lane = lax.iota(jnp.int32, _L)
        for src_hbm, tbl_hbm, out_hbm in ((head_hbm, entp_hbm, out_h),
                                          (rel_hbm, relp_hbm, out_r),
                                          (tail_hbm, entp_hbm, out_t)):
            pltpu.sync_copy(src_hbm.at[pl.ds(base, bw)], idxv)
            for c in range(ng):
                s = pl.ds(c * _L, _L)
                e = idxv[s]
                rowv[s] = (
                    jax.lax.shift_left(
                        jax.lax.shift_right_logical(e, 9), 7)
                    + (e & 127))
            cps = [pltpu.async_copy(
                       tbl_hbm.at[rowv.at[pl.ds(j * 128, 128)]],
                       grows.at[pl.ds(j * 128, 128)], sem)
                   for j in range(4)]
            for cp in cps:
                cp.wait()

            @pl.loop(0, ng)
            def _grp(c):
                s = pl.ds(pl.multiple_of(c * _L, _L), _L)
                off = (jax.lax.shift_right_logical(idxv[s], 7) & 3) * _D
                bvec = c * _L + lane
                cbv = jnp.full((_L,), c // 8, dtype=jnp.int32)
                lpos = (c % 8) * _L + lane
                for kd in range(_D):
                    v = plsc.load_gather(grows, [bvec, off + kd])
                    plsc.store_scatter(
                        stage,
                        [cbv, jnp.full((_L,), kd, dtype=jnp.int32), lpos],
                        v)

            for cb in range(4):
                pltpu.sync_copy(
                    stage.at[cb],
                    out_hbm.at[:, pl.ds(base + cb * 128, 128)])

    return k


def kernel(head, rel, tail, entity_table, relation_table):
    pack_ent, pack_rel = _packers()
    entp = pack_ent(entity_table.T)
    relp = pack_rel(relation_table.T)
    out_h, out_r, out_t = _gather_kernel()(head, rel, tail, entp, relp)
    return (out_h.T, out_r.T, out_t.T)
